# baseline (speedup 1.0000x reference)
import functools

import jax
import jax.numpy as jnp
from jax.experimental import pallas as pl
from jax.experimental.pallas import tpu as pltpu

_ANCHORS = ((116.0, 90.0), (156.0, 198.0), (373.0, 326.0))
_IMG = 512.0
_IGNORE_THR = 0.5
_LXY = 2.5
_LCONF = 5.0
_EPS = 1e-12
_TOP = 1.0 - 1e-12


def _target_build(t_ref, b, in_h, in_w):
    gx = t_ref[b, 0, 1] * in_w
    gy = t_ref[b, 0, 2] * in_h
    gw = t_ref[b, 0, 3] * in_w
    gh = t_ref[b, 0, 4] * in_h
    fx = jnp.floor(gx)
    fy = jnp.floor(gy)
    gi = fx.astype(jnp.int32)
    gj = fy.astype(jnp.int32)
    tx = gx - fx
    ty = gy - fy
    stride_w = _IMG / in_w
    stride_h = _IMG / in_h
    ious = []
    for aw, ah in _ANCHORS:
        aw = aw / stride_w
        ah = ah / stride_h
        inter = (jnp.maximum(jnp.minimum(gw, aw), 0.0)
                 * jnp.maximum(jnp.minimum(gh, ah), 0.0))
        union = gw * gh + aw * ah - inter + 1e-16
        ious.append(inter / union)
    best = jnp.int32(0)
    bv = ious[0]
    best = jnp.where(ious[1] > bv, jnp.int32(1), best)
    bv = jnp.maximum(bv, ious[1])
    best = jnp.where(ious[2] > bv, jnp.int32(2), best)
    return gi, gj, tx, ty, ious, best


def _bce_point(z, t):
    p = jnp.clip(jax.nn.sigmoid(z), _EPS, _TOP)
    return -(t * jnp.log(p) + (1.0 - t) * jnp.log(1.0 - p))


def _body(t_ref, conf_ref, hbm_ref, out_ref, xrow, yrow, xsem, ysem,
          *, in_h, in_w, n_total):
    b = pl.program_id(0)
    a = pl.program_id(1)

    gi, gj, tx, ty, ious, best = _target_build(t_ref, b, in_h, in_w)

    @pl.when(a == 0)
    def _issue():
        ch = 3 * best
        pltpu.make_async_copy(
            hbm_ref.at[b, ch, pl.ds(gj, 1), :], xrow, xsem).start()
        pltpu.make_async_copy(
            hbm_ref.at[b, ch + 1, pl.ds(gj, 1), :], yrow, ysem).start()

    z = conf_ref[0, 0, :, :]
    l_dense = jnp.log1p(jnp.exp(z))
    rows = jax.lax.broadcasted_iota(jnp.int32, (in_h, in_w), 0)
    cols = jax.lax.broadcasted_iota(jnp.int32, (in_h, in_w), 1)
    sel = (rows == gj) & (cols == gi)
    iou = jnp.where(a == 1, ious[1], jnp.where(a == 2, ious[2], ious[0]))
    ignore = iou > _IGNORE_THR
    noobj = jnp.sum(jnp.where(sel & ignore, 0.0, l_dense))
    z_t = jnp.sum(jnp.where(sel, z, 0.0))

    p_t = jnp.clip(jax.nn.sigmoid(z_t), _EPS, _TOP)
    obj = jnp.where(a == best, -jnp.log(p_t), 0.0)

    contrib = 0.5 * _LCONF * noobj + _LCONF * obj

    @pl.when((b == 0) & (a == 0))
    def _init():
        out_ref[0, 0] = 0.0

    out_ref[0, 0] += contrib / n_total

    @pl.when(a == 2)
    def _xy():
        ch = 3 * best
        pltpu.make_async_copy(
            hbm_ref.at[b, ch, pl.ds(gj, 1), :], xrow, xsem).wait()
        pltpu.make_async_copy(
            hbm_ref.at[b, ch + 1, pl.ds(gj, 1), :], yrow, ysem).wait()
        lane = jax.lax.broadcasted_iota(jnp.int32, (1, in_w), 1)
        zx = jnp.sum(jnp.where(lane == gi, xrow[:, :], 0.0))
        zy = jnp.sum(jnp.where(lane == gi, yrow[:, :], 0.0))
        bce = _LXY * (_bce_point(zx, tx) + _bce_point(zy, ty))
        out_ref[0, 0] += bce / n_total


def kernel(input, targets):
    bs, ch, in_h, in_w = input.shape
    n_total = bs * 3 * in_h * in_w

    body = functools.partial(_body, in_h=in_h, in_w=in_w,
                             n_total=float(n_total))
    out = pl.pallas_call(
        body,
        grid=(bs, 3),
        in_specs=[
            pl.BlockSpec(targets.shape, lambda b, a: (0, 0, 0),
                         memory_space=pltpu.SMEM),
            pl.BlockSpec((1, 1, in_h, in_w), lambda b, a: (b, 3 * a + 2, 0, 0)),
            pl.BlockSpec(memory_space=pl.ANY),
        ],
        out_specs=pl.BlockSpec((1, 1), lambda b, a: (0, 0),
                               memory_space=pltpu.SMEM),
        out_shape=jax.ShapeDtypeStruct((1, 1), jnp.float32),
        scratch_shapes=[
            pltpu.VMEM((1, in_w), jnp.float32),
            pltpu.VMEM((1, in_w), jnp.float32),
            pltpu.SemaphoreType.DMA,
            pltpu.SemaphoreType.DMA,
        ],
    )(targets, input, input)
    return out[0, 0]


# speedup vs baseline: 3.4050x; 3.4050x over previous
import functools

import jax
import jax.numpy as jnp
from jax.experimental import pallas as pl
from jax.experimental.pallas import tpu as pltpu

_ANCHORS = ((116.0, 90.0), (156.0, 198.0), (373.0, 326.0))
_IMG = 512.0
_IGNORE_THR = 0.5
_LXY = 2.5
_LCONF = 5.0
_EPS = 1e-12
_TOP = 1.0 - 1e-12


def _row_dmas(t_ref, hbm_ref, rows, sem, bs, in_h):
    copies = []
    for b in range(bs):
        gj = jnp.floor(t_ref[b, 0, 2] * in_h).astype(jnp.int32)
        copies.append(pltpu.make_async_copy(
            hbm_ref.at[b, :, pl.ds(gj, 1), :], rows.at[b], sem))
    return copies


def _body(t_ref, tv_ref, conf_ref, hbm_ref, out_ref, acc, rows, sem,
          *, in_h, in_w, n_total, bs):
    i = pl.program_id(0)
    nsteps = pl.num_programs(0)

    @pl.when(i == 0)
    def _issue():
        for c in _row_dmas(t_ref, hbm_ref, rows, sem, bs, in_h):
            c.start()

    l_dense = jnp.sum(jnp.log1p(jnp.exp(conf_ref[...])), axis=(0, 1))

    @pl.when(i == 0)
    def _init():
        acc[...] = l_dense

    @pl.when(i > 0)
    def _accum():
        acc[...] += l_dense

    @pl.when(i == nsteps - 1)
    def _finish():
        for c in _row_dmas(t_ref, hbm_ref, rows, sem, bs, in_h):
            c.wait()

        tv = tv_ref[:, 0, :]
        gx = tv[:, 1:2] * in_w
        gy = tv[:, 2:3] * in_h
        gw = tv[:, 3:4] * in_w
        gh = tv[:, 4:5] * in_h
        fx = jnp.floor(gx)
        fy = jnp.floor(gy)
        gi = fx.astype(jnp.int32)
        tx = gx - fx
        ty = gy - fy

        stride_w = _IMG / in_w
        stride_h = _IMG / in_h
        ious = []
        for aw, ah in _ANCHORS:
            aw = aw / stride_w
            ah = ah / stride_h
            inter = (jnp.maximum(jnp.minimum(gw, aw), 0.0)
                     * jnp.maximum(jnp.minimum(gh, ah), 0.0))
            union = gw * gh + aw * ah - inter + 1e-16
            ious.append(inter / union)
        best = jnp.zeros_like(gi)
        bv = ious[0]
        best = jnp.where(ious[1] > bv, jnp.int32(1), best)
        bv = jnp.maximum(bv, ious[1])
        best = jnp.where(ious[2] > bv, jnp.int32(2), best)

        v = rows[:, :, 0, :]
        lane = jax.lax.broadcasted_iota(jnp.int32, v.shape, 2)
        vals = jnp.sum(jnp.where(lane == gi[:, :, None], v, 0.0), axis=2)

        sparse = jnp.zeros_like(gx)
        zx = jnp.zeros_like(gx)
        zy = jnp.zeros_like(gx)
        for a in range(3):
            zc = vals[:, 3 * a + 2:3 * a + 3]
            sparse -= jnp.where(ious[a] > _IGNORE_THR,
                                0.5 * _LCONF * jnp.log1p(jnp.exp(zc)), 0.0)
            p_t = jnp.clip(jax.nn.sigmoid(zc), _EPS, _TOP)
            sparse += jnp.where(a == best, -_LCONF * jnp.log(p_t), 0.0)
            zx += jnp.where(a == best, vals[:, 3 * a:3 * a + 1], 0.0)
            zy += jnp.where(a == best, vals[:, 3 * a + 1:3 * a + 2], 0.0)
        for z_v, t_v in ((zx, tx), (zy, ty)):
            p_v = jnp.clip(jax.nn.sigmoid(z_v), _EPS, _TOP)
            sparse += -_LXY * (t_v * jnp.log(p_v)
                               + (1.0 - t_v) * jnp.log(1.0 - p_v))

        total = 0.5 * _LCONF * jnp.sum(acc[...]) + jnp.sum(sparse)
        out_ref[0, 0] = total / n_total


def kernel(input, targets):
    bs, ch, in_h, in_w = input.shape
    bb = 4
    body = functools.partial(_body, in_h=in_h, in_w=in_w,
                             n_total=float(bs * 3 * in_h * in_w), bs=bs)
    out = pl.pallas_call(
        body,
        grid=(3 * bs // bb,),
        in_specs=[
            pl.BlockSpec(targets.shape, lambda i: (0, 0, 0),
                         memory_space=pltpu.SMEM),
            pl.BlockSpec(targets.shape, lambda i: (0, 0, 0)),
            pl.BlockSpec((bb, 1, in_h, in_w),
                         lambda i: (i // 3, (i % 3) * 3 + 2, 0, 0)),
            pl.BlockSpec(memory_space=pl.ANY),
        ],
        out_specs=pl.BlockSpec((1, 1), lambda i: (0, 0),
                               memory_space=pltpu.SMEM),
        out_shape=jax.ShapeDtypeStruct((1, 1), jnp.float32),
        scratch_shapes=[
            pltpu.VMEM((in_h, in_w), jnp.float32),
            pltpu.VMEM((bs, ch, 1, in_w), jnp.float32),
            pltpu.SemaphoreType.DMA,
        ],
    )(targets, targets, input, input)
    return out[0, 0]
